# baseline (speedup 1.0000x reference)
import functools
import jax
import jax.numpy as jnp
from jax import lax
from jax.experimental import pallas as pl
from jax.experimental.pallas import tpu as pltpu, tpu_sc as plsc

B_, A_, C_, NOBJ_ = 16, 24564, 81, 24
A_PAD = 24576
BLK_A = 4096
N_BLK = A_PAD // BLK_A
THRESH = 0.5
NEG_RATIO = 3
V0, V1 = 0.1, 0.2

N_ANCH = B_ * A_PAD
NW = 32
PERW = N_ANCH // NW
CHUNK = 128
NCH = PERW // CHUNK
TBL_ROWS = B_ * A_ * C_ // 16


def _smooth_l1(d):
    ad = jnp.abs(d)
    return jnp.where(ad < 1.0, 0.5 * d * d, ad - 0.5)


def _match_body(anct_ref, lt_ref, ltt_ref, lab_ref, lpt_ref,
                conf_ref, posf_ref, npos_ref, lloss_ref, row_ref):
    anc = anct_ref[...]
    tru = lt_ref[0]
    trt = ltt_ref[0]
    lab = lab_ref[0]
    lp = lpt_ref[0]

    cx, cy, w, h = anc[0:1], anc[1:2], anc[2:3], anc[3:4]
    ax0 = cx - w * 0.5
    ay0 = cy - h * 0.5
    ax1 = cx + w * 0.5
    ay1 = cy + h * 0.5

    tx0, ty0 = tru[:, 0:1], tru[:, 1:2]
    tx1, ty1 = tru[:, 2:3], tru[:, 3:4]

    ix = jnp.clip(jnp.minimum(tx1, ax1) - jnp.maximum(tx0, ax0), 0.0, None)
    iy = jnp.clip(jnp.minimum(ty1, ay1) - jnp.maximum(ty0, ay0), 0.0, None)
    inter = ix * iy
    area_t = (tx1 - tx0) * (ty1 - ty0)
    area_a = (ax1 - ax0) * (ay1 - ay0)
    ov = inter / (area_t + area_a - inter)

    lane = jax.lax.broadcasted_iota(jnp.int32, (1, A_PAD), 1)
    valid = lane < A_
    ov = jnp.where(valid, ov, -1.0)

    ti = jax.lax.broadcasted_iota(jnp.int32, (NOBJ_, A_PAD), 0)
    laneb = jax.lax.broadcasted_iota(jnp.int32, (NOBJ_, A_PAD), 1)

    bto = jnp.max(ov, axis=0, keepdims=True)
    bti = jnp.min(jnp.where(ov >= bto, ti, NOBJ_), axis=0, keepdims=True)

    bpv = jnp.max(ov, axis=1, keepdims=True)
    bpi = jnp.min(jnp.where(ov >= bpv, laneb, A_PAD), axis=1, keepdims=True)

    forced = jnp.max(jnp.where(lane == bpi, ti, -1), axis=0, keepdims=True)
    bti = jnp.where(forced >= 0, forced, bti)
    bto = jnp.where(forced >= 0, 2.0, bto)

    onehot = (bti == ti).astype(jnp.float32)
    coef = jnp.concatenate([trt, lab.astype(jnp.float32),
                            jnp.zeros((3, NOBJ_), jnp.float32)], axis=0)
    g = jax.lax.dot_general(coef, onehot, (((1,), (0,)), ((), ())),
                            preferred_element_type=jnp.float32)
    mx0, my0, mx1, my1, labg = g[0:1], g[1:2], g[2:3], g[3:4], g[4:5]

    conf = jnp.where(bto < THRESH, 0, (labg + 0.5).astype(jnp.int32))
    pos = conf > 0

    b = pl.program_id(0)
    a_real = jnp.minimum(lane, A_ - 1)
    e = (b * A_ + a_real) * C_ + conf
    row_ref[...] = e.reshape(1, 1, A_PAD)

    g_cx = ((mx0 + mx1) * 0.5 - cx) / (V0 * w)
    g_cy = ((my0 + my1) * 0.5 - cy) / (V0 * h)
    sw = jnp.where(pos, (mx1 - mx0) / w, 1.0)
    sh = jnp.where(pos, (my1 - my0) / h, 1.0)
    g_w = jnp.log(sw) / V1
    g_h = jnp.log(sh) / V1

    t = (_smooth_l1(lp[0:1] - g_cx) + _smooth_l1(lp[1:2] - g_cy)
         + _smooth_l1(lp[2:3] - g_w) + _smooth_l1(lp[3:4] - g_h))
    lloss = jnp.sum(jnp.where(pos, t, 0.0))
    npos = jnp.sum(jnp.where(pos, 1, 0))

    conf_ref[...] = conf.reshape(1, 1, A_PAD)
    posf_ref[...] = pos.astype(jnp.float32).reshape(1, 1, A_PAD)
    npos_ref[...] = npos.reshape(1, 1, 1)
    lloss_ref[...] = lloss.reshape(1, 1, 1)


def _se_body(x_ref, se_ref):
    x = x_ref[0]
    ones = jnp.ones((1, C_), jnp.float32)
    dn = (((1,), (1,)), ((), ()))
    se = jax.lax.dot_general(ones, jnp.exp(x), dn,
                             preferred_element_type=jnp.float32)
    se_ref[...] = se.reshape(1, 1, BLK_A)


def _sc_gather_body(tbl_hbm, eidx_hbm, ga_hbm, idx_v, out_v, sem):
    wid = lax.axis_index("s") * 2 + lax.axis_index("c")
    base = wid * PERW
    pltpu.sync_copy(eidx_hbm.at[pl.ds(base, PERW)], idx_v)

    def fire(g, carry):
        pltpu.make_async_copy(
            tbl_hbm.at[idx_v.at[pl.ds(g * CHUNK, CHUNK)]],
            out_v.at[pl.ds(g * CHUNK, CHUNK)], sem).start()
        return carry

    lax.fori_loop(0, NCH, fire, 0)
    pltpu.make_async_copy(ga_hbm.at[pl.ds(base, PERW)], out_v, sem).wait()
    pltpu.sync_copy(out_v, ga_hbm.at[pl.ds(base, PERW)])


def _sc_gather(tbl, eidx):
    fn = functools.partial(
        pl.kernel,
        mesh=plsc.VectorSubcoreMesh(core_axis_name="c",
                                    subcore_axis_name="s"),
        out_type=jax.ShapeDtypeStruct((N_ANCH,), jnp.float32),
        scratch_types=[
            pltpu.VMEM((PERW,), jnp.int32),
            pltpu.VMEM((PERW,), jnp.float32),
            pltpu.SemaphoreType.DMA,
        ],
    )(_sc_gather_body)
    return fn(tbl, eidx)


def _select_body(se_ref, ga_ref, posf_ref, npos_ref, lloss_ref,
                 cl_ref, ll_ref, tot_ref):
    se = se_ref[:, 0, :]
    ga = ga_ref[...]
    pos = posf_ref[:, 0, :] > 0.5
    npos = npos_ref[:, 0, :]

    lane = jax.lax.broadcasted_iota(jnp.int32, (B_, A_PAD), 1)
    valid = lane < A_

    ce = jnp.where(valid, jnp.log(se) - ga, 0.0)
    cls = jnp.maximum(jnp.where(pos, 0.0, ce), 0.0)
    bits = jnp.where(valid, jax.lax.bitcast_convert_type(cls, jnp.int32),
                     -1)

    k = jnp.minimum(NEG_RATIO * npos, A_ - 1)

    def step(_, carry):
        lo, hi = carry
        mid = lo + (hi - lo) // 2
        cnt = jnp.sum(jnp.where(bits >= mid, 1, 0), axis=1, keepdims=True)
        ge = cnt >= k
        return jnp.where(ge, mid, lo), jnp.where(ge, hi, mid)

    lo = jnp.zeros((B_, 1), jnp.int32)
    hi = jnp.full((B_, 1), 0x7F800000, jnp.int32)
    lo, hi = jax.lax.fori_loop(0, 31, step, (lo, hi))

    sel = jnp.logical_or(pos, bits >= lo)
    class_sum = jnp.sum(jnp.where(sel, ce, 0.0))
    nm = jnp.sum(npos).astype(jnp.float32)
    cl = class_sum / nm
    ll = jnp.sum(lloss_ref[...]) / nm
    cl_ref[...] = cl.reshape(1, 1)
    ll_ref[...] = ll.reshape(1, 1)
    tot_ref[...] = (cl + ll).reshape(1, 1)


@jax.jit
def kernel(class_preds, class_targets, loc_preds, loc_targets, anchors):
    f32 = jnp.float32
    anct = jnp.pad(anchors, ((0, A_PAD - A_), (0, 0)),
                   constant_values=1.0).T
    ltt = jnp.transpose(loc_targets, (0, 2, 1))
    lab3 = class_targets.reshape(B_, 1, NOBJ_)
    lpt = jnp.pad(jnp.transpose(loc_preds, (0, 2, 1)),
                  ((0, 0), (0, 0), (0, A_PAD - A_)))

    conf3, posf3, npos3, lloss3, row3 = pl.pallas_call(
        _match_body,
        grid=(B_,),
        in_specs=[
            pl.BlockSpec((4, A_PAD), lambda b: (0, 0)),
            pl.BlockSpec((1, NOBJ_, 4), lambda b: (b, 0, 0)),
            pl.BlockSpec((1, 4, NOBJ_), lambda b: (b, 0, 0)),
            pl.BlockSpec((1, 1, NOBJ_), lambda b: (b, 0, 0)),
            pl.BlockSpec((1, 4, A_PAD), lambda b: (b, 0, 0)),
        ],
        out_specs=[
            pl.BlockSpec((1, 1, A_PAD), lambda b: (b, 0, 0)),
            pl.BlockSpec((1, 1, A_PAD), lambda b: (b, 0, 0)),
            pl.BlockSpec((1, 1, 1), lambda b: (b, 0, 0)),
            pl.BlockSpec((1, 1, 1), lambda b: (b, 0, 0)),
            pl.BlockSpec((1, 1, A_PAD), lambda b: (b, 0, 0)),
        ],
        out_shape=[
            jax.ShapeDtypeStruct((B_, 1, A_PAD), jnp.int32),
            jax.ShapeDtypeStruct((B_, 1, A_PAD), f32),
            jax.ShapeDtypeStruct((B_, 1, 1), jnp.int32),
            jax.ShapeDtypeStruct((B_, 1, 1), f32),
            jax.ShapeDtypeStruct((B_, 1, A_PAD), jnp.int32),
        ],
    )(anct, loc_targets, ltt, lab3, lpt)

    ga_flat = _sc_gather(class_preds.reshape(B_ * A_ * C_),
                         row3.reshape(N_ANCH))

    se3 = pl.pallas_call(
        _se_body,
        grid=(B_, N_BLK),
        in_specs=[
            pl.BlockSpec((1, BLK_A, C_), lambda b, j: (b, j, 0)),
        ],
        out_specs=pl.BlockSpec((1, 1, BLK_A), lambda b, j: (b, 0, j)),
        out_shape=jax.ShapeDtypeStruct((B_, 1, A_PAD), f32),
    )(class_preds)

    cl, ll, tot = pl.pallas_call(
        _select_body,
        in_specs=[
            pl.BlockSpec((B_, 1, A_PAD), lambda: (0, 0, 0)),
            pl.BlockSpec((B_, A_PAD), lambda: (0, 0)),
            pl.BlockSpec((B_, 1, A_PAD), lambda: (0, 0, 0)),
            pl.BlockSpec((B_, 1, 1), lambda: (0, 0, 0)),
            pl.BlockSpec((B_, 1, 1), lambda: (0, 0, 0)),
        ],
        out_specs=[pl.BlockSpec((1, 1), lambda: (0, 0))] * 3,
        out_shape=[jax.ShapeDtypeStruct((1, 1), f32)] * 3,
    )(se3, ga_flat.reshape(B_, A_PAD), posf3, npos3, lloss3)

    return (cl[0, 0], ll[0, 0], tot[0, 0])


# speedup vs baseline: 4.4903x; 4.4903x over previous
import functools
import jax
import jax.numpy as jnp
from jax import lax
from jax.experimental import pallas as pl
from jax.experimental.pallas import tpu as pltpu, tpu_sc as plsc

B_, A_, C_, NOBJ_ = 16, 24564, 81, 24
A_PAD = 24576
BLK_A = 4096
N_BLK = A_PAD // BLK_A
THRESH = 0.5
NEG_RATIO = 3
V0, V1 = 0.1, 0.2

N_ANCH = B_ * A_PAD
NW = 32
PERW = N_ANCH // NW
CHUNK = 128
NCH = PERW // CHUNK
TBL_ROWS = B_ * A_ * C_ // 16


def _smooth_l1(d):
    ad = jnp.abs(d)
    return jnp.where(ad < 1.0, 0.5 * d * d, ad - 0.5)


def _match_body(anct_ref, lt_ref, ltt_ref, lab_ref, lpt_ref,
                conf_ref, posf_ref, npos_ref, lloss_ref):
    anc = anct_ref[...]
    tru = lt_ref[0]
    trt = ltt_ref[0]
    lab = lab_ref[0]
    lp = lpt_ref[0]

    cx, cy, w, h = anc[0:1], anc[1:2], anc[2:3], anc[3:4]
    ax0 = cx - w * 0.5
    ay0 = cy - h * 0.5
    ax1 = cx + w * 0.5
    ay1 = cy + h * 0.5

    tx0, ty0 = tru[:, 0:1], tru[:, 1:2]
    tx1, ty1 = tru[:, 2:3], tru[:, 3:4]

    ix = jnp.clip(jnp.minimum(tx1, ax1) - jnp.maximum(tx0, ax0), 0.0, None)
    iy = jnp.clip(jnp.minimum(ty1, ay1) - jnp.maximum(ty0, ay0), 0.0, None)
    inter = ix * iy
    area_t = (tx1 - tx0) * (ty1 - ty0)
    area_a = (ax1 - ax0) * (ay1 - ay0)
    ov = inter / (area_t + area_a - inter)

    lane = jax.lax.broadcasted_iota(jnp.int32, (1, A_PAD), 1)
    valid = lane < A_
    ov = jnp.where(valid, ov, -1.0)

    ti = jax.lax.broadcasted_iota(jnp.int32, (NOBJ_, A_PAD), 0)
    laneb = jax.lax.broadcasted_iota(jnp.int32, (NOBJ_, A_PAD), 1)

    bto = jnp.max(ov, axis=0, keepdims=True)
    bti = jnp.min(jnp.where(ov >= bto, ti, NOBJ_), axis=0, keepdims=True)

    bpv = jnp.max(ov, axis=1, keepdims=True)
    bpi = jnp.min(jnp.where(ov >= bpv, laneb, A_PAD), axis=1, keepdims=True)

    forced = jnp.max(jnp.where(lane == bpi, ti, -1), axis=0, keepdims=True)
    bti = jnp.where(forced >= 0, forced, bti)
    bto = jnp.where(forced >= 0, 2.0, bto)

    onehot = (bti == ti).astype(jnp.float32)
    coef = jnp.concatenate([trt, lab.astype(jnp.float32),
                            jnp.zeros((3, NOBJ_), jnp.float32)], axis=0)
    g = jax.lax.dot_general(coef, onehot, (((1,), (0,)), ((), ())),
                            preferred_element_type=jnp.float32)
    mx0, my0, mx1, my1, labg = g[0:1], g[1:2], g[2:3], g[3:4], g[4:5]

    conf = jnp.where(bto < THRESH, 0, (labg + 0.5).astype(jnp.int32))
    pos = conf > 0

    g_cx = ((mx0 + mx1) * 0.5 - cx) / (V0 * w)
    g_cy = ((my0 + my1) * 0.5 - cy) / (V0 * h)
    sw = jnp.where(pos, (mx1 - mx0) / w, 1.0)
    sh = jnp.where(pos, (my1 - my0) / h, 1.0)
    g_w = jnp.log(sw) / V1
    g_h = jnp.log(sh) / V1

    t = (_smooth_l1(lp[0:1] - g_cx) + _smooth_l1(lp[1:2] - g_cy)
         + _smooth_l1(lp[2:3] - g_w) + _smooth_l1(lp[3:4] - g_h))
    lloss = jnp.sum(jnp.where(pos, t, 0.0))
    npos = jnp.sum(jnp.where(pos, 1, 0))

    conf_ref[...] = conf.reshape(1, 1, A_PAD)
    posf_ref[...] = pos.astype(jnp.float32).reshape(1, 1, A_PAD)
    npos_ref[...] = npos.reshape(1, 1, 1)
    lloss_ref[...] = lloss.reshape(1, 1, 1)


def _se_body(x_ref, cf_ref, se_ref, ga_ref):
    x = x_ref[0]
    cfr = cf_ref[0]
    cfc = jnp.transpose(cfr)
    ones = jnp.ones((1, C_), jnp.float32)
    dn = (((1,), (1,)), ((), ()))
    se = jax.lax.dot_general(ones, jnp.exp(x), dn,
                             preferred_element_type=jnp.float32)
    ci = jax.lax.broadcasted_iota(jnp.int32, (BLK_A, C_), 1)
    xm = jnp.where(ci == cfc, x, 0.0)
    ga = jax.lax.dot_general(ones, xm, dn,
                             preferred_element_type=jnp.float32)
    se_ref[...] = se.reshape(1, 1, BLK_A)
    ga_ref[...] = ga.reshape(1, 1, BLK_A)


def _select_body(se_ref, ga_ref, posf_ref, npos_ref, lloss_ref,
                 cl_ref, ll_ref, tot_ref):
    se = se_ref[:, 0, :]
    ga = ga_ref[:, 0, :]
    pos = posf_ref[:, 0, :] > 0.5
    npos = npos_ref[:, 0, :]

    lane = jax.lax.broadcasted_iota(jnp.int32, (B_, A_PAD), 1)
    valid = lane < A_

    ce = jnp.where(valid, jnp.log(se) - ga, 0.0)
    cls = jnp.maximum(jnp.where(pos, 0.0, ce), 0.0)
    bits = jnp.where(valid, jax.lax.bitcast_convert_type(cls, jnp.int32),
                     -1)

    k = jnp.minimum(NEG_RATIO * npos, A_ - 1)

    def step(_, carry):
        lo, hi = carry
        mid = lo + (hi - lo) // 2
        cnt = jnp.sum(jnp.where(bits >= mid, 1, 0), axis=1, keepdims=True)
        ge = cnt >= k
        return jnp.where(ge, mid, lo), jnp.where(ge, hi, mid)

    lo = jnp.zeros((B_, 1), jnp.int32)
    hi = jnp.full((B_, 1), 0x7F800000, jnp.int32)
    lo, hi = jax.lax.fori_loop(0, 31, step, (lo, hi))

    sel = jnp.logical_or(pos, bits >= lo)
    class_sum = jnp.sum(jnp.where(sel, ce, 0.0))
    nm = jnp.sum(npos).astype(jnp.float32)
    cl = class_sum / nm
    ll = jnp.sum(lloss_ref[...]) / nm
    cl_ref[...] = cl.reshape(1, 1)
    ll_ref[...] = ll.reshape(1, 1)
    tot_ref[...] = (cl + ll).reshape(1, 1)


@jax.jit
def kernel(class_preds, class_targets, loc_preds, loc_targets, anchors):
    f32 = jnp.float32
    anct = jnp.pad(anchors, ((0, A_PAD - A_), (0, 0)),
                   constant_values=1.0).T
    ltt = jnp.transpose(loc_targets, (0, 2, 1))
    lab3 = class_targets.reshape(B_, 1, NOBJ_)
    lpt = jnp.pad(jnp.transpose(loc_preds, (0, 2, 1)),
                  ((0, 0), (0, 0), (0, A_PAD - A_)))

    conf3, posf3, npos3, lloss3 = pl.pallas_call(
        _match_body,
        grid=(B_,),
        in_specs=[
            pl.BlockSpec((4, A_PAD), lambda b: (0, 0)),
            pl.BlockSpec((1, NOBJ_, 4), lambda b: (b, 0, 0)),
            pl.BlockSpec((1, 4, NOBJ_), lambda b: (b, 0, 0)),
            pl.BlockSpec((1, 1, NOBJ_), lambda b: (b, 0, 0)),
            pl.BlockSpec((1, 4, A_PAD), lambda b: (b, 0, 0)),
        ],
        out_specs=[
            pl.BlockSpec((1, 1, A_PAD), lambda b: (b, 0, 0)),
            pl.BlockSpec((1, 1, A_PAD), lambda b: (b, 0, 0)),
            pl.BlockSpec((1, 1, 1), lambda b: (b, 0, 0)),
            pl.BlockSpec((1, 1, 1), lambda b: (b, 0, 0)),
        ],
        out_shape=[
            jax.ShapeDtypeStruct((B_, 1, A_PAD), jnp.int32),
            jax.ShapeDtypeStruct((B_, 1, A_PAD), f32),
            jax.ShapeDtypeStruct((B_, 1, 1), jnp.int32),
            jax.ShapeDtypeStruct((B_, 1, 1), f32),
        ],
    )(anct, loc_targets, ltt, lab3, lpt)

    se3, ga3 = pl.pallas_call(
        _se_body,
        grid=(B_, N_BLK),
        in_specs=[
            pl.BlockSpec((1, BLK_A, C_), lambda b, j: (b, j, 0)),
            pl.BlockSpec((1, 1, BLK_A), lambda b, j: (b, 0, j)),
        ],
        out_specs=[
            pl.BlockSpec((1, 1, BLK_A), lambda b, j: (b, 0, j)),
            pl.BlockSpec((1, 1, BLK_A), lambda b, j: (b, 0, j)),
        ],
        out_shape=[
            jax.ShapeDtypeStruct((B_, 1, A_PAD), f32),
            jax.ShapeDtypeStruct((B_, 1, A_PAD), f32),
        ],
    )(class_preds, conf3)

    cl, ll, tot = pl.pallas_call(
        _select_body,
        in_specs=[
            pl.BlockSpec((B_, 1, A_PAD), lambda: (0, 0, 0)),
            pl.BlockSpec((B_, 1, A_PAD), lambda: (0, 0, 0)),
            pl.BlockSpec((B_, 1, A_PAD), lambda: (0, 0, 0)),
            pl.BlockSpec((B_, 1, 1), lambda: (0, 0, 0)),
            pl.BlockSpec((B_, 1, 1), lambda: (0, 0, 0)),
        ],
        out_specs=[pl.BlockSpec((1, 1), lambda: (0, 0))] * 3,
        out_shape=[jax.ShapeDtypeStruct((1, 1), f32)] * 3,
    )(se3, ga3, posf3, npos3, lloss3)

    return (cl[0, 0], ll[0, 0], tot[0, 0])


# speedup vs baseline: 4.7825x; 1.0651x over previous
import functools
import jax
import jax.numpy as jnp
from jax import lax
from jax.experimental import pallas as pl
from jax.experimental.pallas import tpu as pltpu, tpu_sc as plsc

B_, A_, C_, NOBJ_ = 16, 24564, 81, 24
A_PAD = 24576
BLK_A = 8192
N_BLK = A_PAD // BLK_A
THRESH = 0.5
NEG_RATIO = 3
V0, V1 = 0.1, 0.2

N_ANCH = B_ * A_PAD
NW = 32
PERW = N_ANCH // NW
CHUNK = 128
NCH = PERW // CHUNK
TBL_ROWS = B_ * A_ * C_ // 16


def _smooth_l1(d):
    ad = jnp.abs(d)
    return jnp.where(ad < 1.0, 0.5 * d * d, ad - 0.5)


def _match_body(anct_ref, lt_ref, ltt_ref, lab_ref, lpt_ref,
                conf_ref, posf_ref, npos_ref, lloss_ref):
    anc = anct_ref[...]
    tru = lt_ref[0]
    trt = ltt_ref[0]
    lab = lab_ref[0]
    lp = lpt_ref[0]

    cx, cy, w, h = anc[0:1], anc[1:2], anc[2:3], anc[3:4]
    ax0 = cx - w * 0.5
    ay0 = cy - h * 0.5
    ax1 = cx + w * 0.5
    ay1 = cy + h * 0.5

    tx0, ty0 = tru[:, 0:1], tru[:, 1:2]
    tx1, ty1 = tru[:, 2:3], tru[:, 3:4]

    ix = jnp.clip(jnp.minimum(tx1, ax1) - jnp.maximum(tx0, ax0), 0.0, None)
    iy = jnp.clip(jnp.minimum(ty1, ay1) - jnp.maximum(ty0, ay0), 0.0, None)
    inter = ix * iy
    area_t = (tx1 - tx0) * (ty1 - ty0)
    area_a = (ax1 - ax0) * (ay1 - ay0)
    ov = inter / (area_t + area_a - inter)

    lane = jax.lax.broadcasted_iota(jnp.int32, (1, A_PAD), 1)
    valid = lane < A_
    ov = jnp.where(valid, ov, -1.0)

    ti = jax.lax.broadcasted_iota(jnp.int32, (NOBJ_, A_PAD), 0)
    laneb = jax.lax.broadcasted_iota(jnp.int32, (NOBJ_, A_PAD), 1)

    bto = jnp.max(ov, axis=0, keepdims=True)
    bti = jnp.min(jnp.where(ov >= bto, ti, NOBJ_), axis=0, keepdims=True)

    bpv = jnp.max(ov, axis=1, keepdims=True)
    bpi = jnp.min(jnp.where(ov >= bpv, laneb, A_PAD), axis=1, keepdims=True)

    forced = jnp.max(jnp.where(lane == bpi, ti, -1), axis=0, keepdims=True)
    bti = jnp.where(forced >= 0, forced, bti)
    bto = jnp.where(forced >= 0, 2.0, bto)

    onehot = (bti == ti).astype(jnp.float32)
    coef = jnp.concatenate([trt, lab.astype(jnp.float32),
                            jnp.zeros((3, NOBJ_), jnp.float32)], axis=0)
    g = jax.lax.dot_general(coef, onehot, (((1,), (0,)), ((), ())),
                            preferred_element_type=jnp.float32)
    mx0, my0, mx1, my1, labg = g[0:1], g[1:2], g[2:3], g[3:4], g[4:5]

    conf = jnp.where(bto < THRESH, 0, (labg + 0.5).astype(jnp.int32))
    pos = conf > 0

    g_cx = ((mx0 + mx1) * 0.5 - cx) / (V0 * w)
    g_cy = ((my0 + my1) * 0.5 - cy) / (V0 * h)
    sw = jnp.where(pos, (mx1 - mx0) / w, 1.0)
    sh = jnp.where(pos, (my1 - my0) / h, 1.0)
    g_w = jnp.log(sw) / V1
    g_h = jnp.log(sh) / V1

    t = (_smooth_l1(lp[0:1] - g_cx) + _smooth_l1(lp[1:2] - g_cy)
         + _smooth_l1(lp[2:3] - g_w) + _smooth_l1(lp[3:4] - g_h))
    lloss = jnp.sum(jnp.where(pos, t, 0.0))
    npos = jnp.sum(jnp.where(pos, 1, 0))

    conf_ref[...] = conf.reshape(1, 1, A_PAD)
    posf_ref[...] = pos.astype(jnp.float32).reshape(1, 1, A_PAD)
    npos_ref[...] = npos.reshape(1, 1, 1)
    lloss_ref[...] = lloss.reshape(1, 1, 1)


def _se_body(x_ref, cf_ref, se_ref, ga_ref):
    x = x_ref[0]
    cfr = cf_ref[0]
    cfc = jnp.transpose(cfr)
    ones = jnp.ones((1, C_), jnp.float32)
    dn = (((1,), (1,)), ((), ()))
    se = jax.lax.dot_general(ones, jnp.exp(x), dn,
                             preferred_element_type=jnp.float32)
    ci = jax.lax.broadcasted_iota(jnp.int32, (BLK_A, C_), 1)
    xm = jnp.where(ci == cfc, x, 0.0)
    ga = jax.lax.dot_general(ones, xm, dn,
                             preferred_element_type=jnp.float32)
    se_ref[...] = se.reshape(1, 1, BLK_A)
    ga_ref[...] = ga.reshape(1, 1, BLK_A)


def _select_body(se_ref, ga_ref, posf_ref, npos_ref, lloss_ref,
                 cl_ref, ll_ref, tot_ref):
    se = se_ref[:, 0, :]
    ga = ga_ref[:, 0, :]
    pos = posf_ref[:, 0, :] > 0.5
    npos = npos_ref[:, 0, :]

    lane = jax.lax.broadcasted_iota(jnp.int32, (B_, A_PAD), 1)
    valid = lane < A_

    ce = jnp.where(valid, jnp.log(se) - ga, 0.0)
    cls = jnp.maximum(jnp.where(pos, 0.0, ce), 0.0)
    bits = jnp.where(valid, jax.lax.bitcast_convert_type(cls, jnp.int32),
                     -1)

    k = jnp.minimum(NEG_RATIO * npos, A_ - 1)

    def step(_, carry):
        lo, hi = carry
        mid = lo + (hi - lo) // 2
        cnt = jnp.sum(jnp.where(bits >= mid, 1, 0), axis=1, keepdims=True)
        ge = cnt >= k
        return jnp.where(ge, mid, lo), jnp.where(ge, hi, mid)

    lo = jnp.zeros((B_, 1), jnp.int32)
    hi = jnp.full((B_, 1), 0x7F800000, jnp.int32)
    lo, hi = jax.lax.fori_loop(0, 31, step, (lo, hi))

    sel = jnp.logical_or(pos, bits >= lo)
    class_sum = jnp.sum(jnp.where(sel, ce, 0.0))
    nm = jnp.sum(npos).astype(jnp.float32)
    cl = class_sum / nm
    ll = jnp.sum(lloss_ref[...]) / nm
    cl_ref[...] = cl.reshape(1, 1)
    ll_ref[...] = ll.reshape(1, 1)
    tot_ref[...] = (cl + ll).reshape(1, 1)


@jax.jit
def kernel(class_preds, class_targets, loc_preds, loc_targets, anchors):
    f32 = jnp.float32
    anct = jnp.pad(anchors, ((0, A_PAD - A_), (0, 0)),
                   constant_values=1.0).T
    ltt = jnp.transpose(loc_targets, (0, 2, 1))
    lab3 = class_targets.reshape(B_, 1, NOBJ_)
    lpt = jnp.pad(jnp.transpose(loc_preds, (0, 2, 1)),
                  ((0, 0), (0, 0), (0, A_PAD - A_)))

    conf3, posf3, npos3, lloss3 = pl.pallas_call(
        _match_body,
        grid=(B_,),
        in_specs=[
            pl.BlockSpec((4, A_PAD), lambda b: (0, 0)),
            pl.BlockSpec((1, NOBJ_, 4), lambda b: (b, 0, 0)),
            pl.BlockSpec((1, 4, NOBJ_), lambda b: (b, 0, 0)),
            pl.BlockSpec((1, 1, NOBJ_), lambda b: (b, 0, 0)),
            pl.BlockSpec((1, 4, A_PAD), lambda b: (b, 0, 0)),
        ],
        out_specs=[
            pl.BlockSpec((1, 1, A_PAD), lambda b: (b, 0, 0)),
            pl.BlockSpec((1, 1, A_PAD), lambda b: (b, 0, 0)),
            pl.BlockSpec((1, 1, 1), lambda b: (b, 0, 0)),
            pl.BlockSpec((1, 1, 1), lambda b: (b, 0, 0)),
        ],
        out_shape=[
            jax.ShapeDtypeStruct((B_, 1, A_PAD), jnp.int32),
            jax.ShapeDtypeStruct((B_, 1, A_PAD), f32),
            jax.ShapeDtypeStruct((B_, 1, 1), jnp.int32),
            jax.ShapeDtypeStruct((B_, 1, 1), f32),
        ],
    )(anct, loc_targets, ltt, lab3, lpt)

    se3, ga3 = pl.pallas_call(
        _se_body,
        grid=(B_, N_BLK),
        in_specs=[
            pl.BlockSpec((1, BLK_A, C_), lambda b, j: (b, j, 0)),
            pl.BlockSpec((1, 1, BLK_A), lambda b, j: (b, 0, j)),
        ],
        out_specs=[
            pl.BlockSpec((1, 1, BLK_A), lambda b, j: (b, 0, j)),
            pl.BlockSpec((1, 1, BLK_A), lambda b, j: (b, 0, j)),
        ],
        out_shape=[
            jax.ShapeDtypeStruct((B_, 1, A_PAD), f32),
            jax.ShapeDtypeStruct((B_, 1, A_PAD), f32),
        ],
    )(class_preds, conf3)

    cl, ll, tot = pl.pallas_call(
        _select_body,
        in_specs=[
            pl.BlockSpec((B_, 1, A_PAD), lambda: (0, 0, 0)),
            pl.BlockSpec((B_, 1, A_PAD), lambda: (0, 0, 0)),
            pl.BlockSpec((B_, 1, A_PAD), lambda: (0, 0, 0)),
            pl.BlockSpec((B_, 1, 1), lambda: (0, 0, 0)),
            pl.BlockSpec((B_, 1, 1), lambda: (0, 0, 0)),
        ],
        out_specs=[pl.BlockSpec((1, 1), lambda: (0, 0))] * 3,
        out_shape=[jax.ShapeDtypeStruct((1, 1), f32)] * 3,
    )(se3, ga3, posf3, npos3, lloss3)

    return (cl[0, 0], ll[0, 0], tot[0, 0])
